# baseline (speedup 1.0000x reference)
import functools

import jax
import jax.numpy as jnp
from jax import lax
from jax.experimental import pallas as pl
from jax.experimental.pallas import tpu as pltpu
from jax.experimental.pallas import tpu_sc as plsc


_B, _N, _IN, _HID, _OUT = 4, 10000, 128, 256, 128


def _sc_gather(emb2d, ns, nid1):
    mesh = plsc.VectorSubcoreMesh(core_axis_name="c", subcore_axis_name="s",
                                  num_cores=1)

    @functools.partial(
        pl.kernel,
        mesh=mesh,
        out_type=(
            jax.ShapeDtypeStruct((_B * 64, _IN), jnp.float32),
            jax.ShapeDtypeStruct((_B, _IN), jnp.float32),
        ),
        scratch_types=[
            pltpu.VMEM((16,), jnp.int32),
            pltpu.VMEM((16,), jnp.int32),
            pltpu.VMEM((16, _IN), jnp.float32),
            pltpu.VMEM((16,), jnp.int32),
            pltpu.SemaphoreType.DMA,
        ],
    )
    def k(emb_hbm, ns_hbm, nid_hbm, ne_hbm, ce_hbm,
          nsv, idx_v, rows_v, nid_v, sem0):
        wid = lax.axis_index("s")

        b = wid // 4
        part = wid % 4
        pltpu.sync_copy(ns_hbm.at[pl.ds(part * 16, 16)], nsv)
        idx_v[:] = nsv[:] + b * _N
        pltpu.async_copy(emb_hbm.at[idx_v], rows_v, sem0).wait()
        pltpu.sync_copy(rows_v, ne_hbm.at[pl.ds(wid * 16, 16)])

        @pl.when(wid == 0)
        def _():
            pltpu.sync_copy(nid_hbm, nid_v.at[pl.ds(0, 1)])
            nid = nid_v[:][0]
            iota16 = lax.iota(jnp.int32, 16)
            idx_v[:] = jnp.minimum(iota16, _B - 1) * _N + nid
            pltpu.async_copy(emb_hbm.at[idx_v], rows_v, sem0).wait()
            pltpu.sync_copy(rows_v.at[pl.ds(0, _B)], ce_hbm)

    return k(emb2d, ns, nid1)


def _leaky(x):
    return jnp.where(x >= 0, x, 0.3 * x)


def _tc_dense_body(ns_ref, nid_ref, w_hbm, nsv_ref, ne_ref, ce_ref, q_ref,
                   qb_ref, wk_ref, wb_ref, o_ref, wrows_v, sem):
    nid = nid_ref[0]
    col0 = pl.multiple_of((nid // 128) * 128, 128)
    lane = nid - col0
    for i in range(64):
        row0 = pl.multiple_of((ns_ref[i] // 8) * 8, 8)
        pltpu.make_async_copy(
            w_hbm.at[pl.ds(row0, 8), pl.ds(col0, 128)],
            wrows_v.at[i],
            sem,
        ).start()

    q = q_ref[:]
    qb = qb_ref[:]
    hs = []
    for b in range(_B):
        ne_b = ne_ref[b * 64:(b + 1) * 64, :]
        hs.append(_leaky(jnp.dot(ne_b, q,
                                 preferred_element_type=jnp.float32,
                                 precision=lax.Precision.HIGHEST)
                         + qb[None, :]))

    for i in range(64):
        pltpu.make_async_copy(
            w_hbm.at[pl.ds(0, 8), pl.ds(col0, 128)],
            wrows_v.at[i],
            sem,
        ).wait()
    subl = nsv_ref[:] % 8
    sel = ((lax.broadcasted_iota(jnp.int32, (64, 8, 128), 1)
            == subl[:, None, None])
           & (lax.broadcasted_iota(jnp.int32, (64, 8, 128), 2) == lane))
    w64 = jnp.sum(jnp.where(sel, wrows_v[:], 0.0), axis=(1, 2))
    denom = jnp.sum(w64) + 1e-6

    ws_rows = [jnp.sum(h * w64[:, None], axis=0, keepdims=True) / denom
               for h in hs]
    wsm = jnp.concatenate(ws_rows, axis=0)
    cc = jnp.concatenate([ce_ref[:], wsm], axis=1)
    h2 = _leaky(jnp.dot(cc, wk_ref[:],
                        preferred_element_type=jnp.float32,
                        precision=lax.Precision.HIGHEST) + wb_ref[:][None, :])
    nrm = jnp.sqrt(jnp.sum(h2 * h2, axis=1, keepdims=True)) + 1e-6
    o_ref[:] = h2 / nrm


def _tc_dense(ns, nid1, weights, ne, ce, Q_kernel, Q_bias, W_kernel, W_bias):
    vmem = pl.BlockSpec(memory_space=pltpu.MemorySpace.VMEM)
    return pl.pallas_call(
        _tc_dense_body,
        in_specs=[
            pl.BlockSpec(memory_space=pltpu.MemorySpace.SMEM),
            pl.BlockSpec(memory_space=pltpu.MemorySpace.SMEM),
            pl.BlockSpec(memory_space=pltpu.MemorySpace.HBM),
            vmem, vmem, vmem, vmem, vmem, vmem, vmem,
        ],
        out_specs=vmem,
        out_shape=jax.ShapeDtypeStruct((_B, _OUT), jnp.float32),
        scratch_shapes=[
            pltpu.VMEM((64, 8, 128), jnp.float32),
            pltpu.SemaphoreType.DMA,
        ],
    )(ns, nid1, weights, ns, ne, ce, Q_kernel, Q_bias, W_kernel, W_bias)


def kernel(embeddings, weights, Q_kernel, Q_bias, W_kernel, W_bias,
           neighbor_set, node_id):
    B, N, IN = embeddings.shape
    ns = neighbor_set.astype(jnp.int32)
    nid1 = jnp.asarray(node_id, jnp.int32).reshape(1)
    emb2d = embeddings.reshape(B * N, IN)
    ne, ce = _sc_gather(emb2d, ns, nid1)
    return _tc_dense(ns, nid1, weights, ne, ce,
                     Q_kernel, Q_bias, W_kernel, W_bias)


# speedup vs baseline: 4.7170x; 4.7170x over previous
import jax
import jax.numpy as jnp
from jax import lax
from jax.experimental import pallas as pl
from jax.experimental.pallas import tpu as pltpu


_B, _N, _IN, _HID, _OUT = 4, 10000, 128, 256, 128


def _leaky(x):
    return jnp.where(x >= 0, x, 0.3 * x)


def _dot(a, b):
    return jnp.dot(a, b, preferred_element_type=jnp.float32)


def _body(ns_ref, nid_ref, emb_hbm, w_hbm, nsv_ref, q_ref, qb_ref,
          wk_ref, wb_ref, o_ref, erows_v, crow_v, wrows_v,
          sem_e, sem_c, sem_w):
    nid = nid_ref[0]
    col0 = pl.multiple_of((nid // 128) * 128, 128)
    lane = nid - col0
    nrow0 = pl.multiple_of((nid // 8) * 8, 8)

    for i in range(64):
        row0 = pl.multiple_of((ns_ref[i] // 8) * 8, 8)
        pltpu.make_async_copy(
            emb_hbm.at[:, pl.ds(row0, 8), :], erows_v.at[i], sem_e,
        ).start()
    pltpu.make_async_copy(
        emb_hbm.at[:, pl.ds(nrow0, 8), :], crow_v, sem_c,
    ).start()
    for i in range(64):
        row0 = pl.multiple_of((ns_ref[i] // 8) * 8, 8)
        pltpu.make_async_copy(
            w_hbm.at[pl.ds(row0, 8), pl.ds(col0, 128)], wrows_v.at[i], sem_w,
        ).start()

    q = q_ref[:]
    qb = qb_ref[:]

    subl = nsv_ref[:] % 8
    colv = lax.broadcasted_iota(jnp.int32, (64, 512), 1)
    rowv = lax.broadcasted_iota(jnp.int32, (64, 512), 0)
    psel = ((colv // 8 == rowv) & (colv % 8 == subl[:, None])
            ).astype(jnp.float32)

    for i in range(64):
        pltpu.make_async_copy(
            emb_hbm.at[:, pl.ds(0, 8), :], erows_v.at[i], sem_e,
        ).wait()

    hs = []
    for b in range(_B):
        er_b = erows_v[:, b, :, :].reshape(512, _IN)
        ne_b = _dot(psel, er_b)
        hs.append(_leaky(_dot(ne_b, q) + qb[None, :]))

    for i in range(64):
        pltpu.make_async_copy(
            w_hbm.at[pl.ds(0, 8), pl.ds(col0, 128)], wrows_v.at[i], sem_w,
        ).wait()
    lsel = (lax.broadcasted_iota(jnp.int32, (128, 8), 0) == lane
            ).astype(jnp.float32)
    wl = _dot(wrows_v[:].reshape(512, 128), lsel)
    w64 = _dot(psel, wl[:, :1])
    denom = jnp.sum(w64) + 1e-6

    pltpu.make_async_copy(
        emb_hbm.at[:, pl.ds(0, 8), :], crow_v, sem_c,
    ).wait()
    csel = (lax.broadcasted_iota(jnp.int32, (_B, 8, 128), 1) == (nid - nrow0))
    ce = jnp.sum(jnp.where(csel, crow_v[:], 0.0), axis=1)

    ws_rows = [jnp.sum(h * w64, axis=0, keepdims=True) / denom
               for h in hs]
    wsm = jnp.concatenate(ws_rows, axis=0)
    cc = jnp.concatenate([ce, wsm], axis=1)
    h2 = _leaky(_dot(cc, wk_ref[:]) + wb_ref[:][None, :])
    nrm = jnp.sqrt(jnp.sum(h2 * h2, axis=1, keepdims=True)) + 1e-6
    o_ref[:] = h2 / nrm


def kernel(embeddings, weights, Q_kernel, Q_bias, W_kernel, W_bias,
           neighbor_set, node_id):
    ns = neighbor_set.astype(jnp.int32)
    nid1 = jnp.asarray(node_id, jnp.int32).reshape(1)
    vmem = pl.BlockSpec(memory_space=pltpu.MemorySpace.VMEM)
    hbm = pl.BlockSpec(memory_space=pltpu.MemorySpace.HBM)
    smem = pl.BlockSpec(memory_space=pltpu.MemorySpace.SMEM)
    return pl.pallas_call(
        _body,
        in_specs=[smem, smem, hbm, hbm, vmem, vmem, vmem, vmem, vmem],
        out_specs=vmem,
        out_shape=jax.ShapeDtypeStruct((_B, _OUT), jnp.float32),
        scratch_shapes=[
            pltpu.VMEM((64, _B, 8, _IN), jnp.float32),
            pltpu.VMEM((_B, 8, _IN), jnp.float32),
            pltpu.VMEM((64, 8, 128), jnp.float32),
            pltpu.SemaphoreType.DMA,
            pltpu.SemaphoreType.DMA,
            pltpu.SemaphoreType.DMA,
        ],
    )(ns, nid1, embeddings, weights, ns, Q_kernel, Q_bias, W_kernel, W_bias)


# speedup vs baseline: 5.0233x; 1.0649x over previous
import jax
import jax.numpy as jnp
from jax import lax
from jax.experimental import pallas as pl
from jax.experimental.pallas import tpu as pltpu


_B, _N, _IN, _HID, _OUT = 4, 10000, 128, 256, 128


def _leaky(x):
    return jnp.where(x >= 0, x, 0.3 * x)


def _dot(a, b):
    return jnp.dot(a, b, preferred_element_type=jnp.float32)


def _body(ns_ref, nid_ref, emb_hbm, w_hbm, ns_hbm, q_hbm, qb_hbm,
          wk_hbm, wb_hbm, o_ref, erows_v, crow_v, wrows_v,
          nsv_v, q_v, qb_v, wk_v, wb_v,
          sem_e, sem_c, sem_w, sem_n, sem_q, sem_qb, sem_wk, sem_wb):
    nid = nid_ref[0]
    col0 = pl.multiple_of((nid // 128) * 128, 128)
    lane = nid - col0
    nrow0 = pl.multiple_of((nid // 8) * 8, 8)

    cp_ns = pltpu.make_async_copy(ns_hbm, nsv_v, sem_n)
    cp_ns.start()
    for i in range(64):
        row0 = pl.multiple_of((ns_ref[i] // 8) * 8, 8)
        pltpu.make_async_copy(
            emb_hbm.at[:, pl.ds(row0, 8), :], erows_v.at[i], sem_e,
        ).start()
    pltpu.make_async_copy(
        emb_hbm.at[:, pl.ds(nrow0, 8), :], crow_v, sem_c,
    ).start()
    for i in range(64):
        row0 = pl.multiple_of((ns_ref[i] // 8) * 8, 8)
        pltpu.make_async_copy(
            w_hbm.at[pl.ds(row0, 8), pl.ds(col0, 128)], wrows_v.at[i], sem_w,
        ).start()
    cp_q = pltpu.make_async_copy(q_hbm, q_v, sem_q)
    cp_qb = pltpu.make_async_copy(qb_hbm, qb_v, sem_qb)
    cp_wk = pltpu.make_async_copy(wk_hbm, wk_v, sem_wk)
    cp_wb = pltpu.make_async_copy(wb_hbm, wb_v, sem_wb)
    cp_q.start(); cp_qb.start(); cp_wk.start(); cp_wb.start()

    cp_ns.wait()
    subl = nsv_v[:] % 8
    colv = lax.broadcasted_iota(jnp.int32, (64, 512), 1)
    rowv = lax.broadcasted_iota(jnp.int32, (64, 512), 0)
    psel = ((colv // 8 == rowv) & (colv % 8 == subl[:, None])
            ).astype(jnp.float32)

    for i in range(64):
        pltpu.make_async_copy(
            emb_hbm.at[:, pl.ds(0, 8), :], erows_v.at[i], sem_e,
        ).wait()
    cp_q.wait()
    cp_qb.wait()
    q = q_v[:]
    qb = qb_v[:]

    hs = []
    for b in range(_B):
        er_b = erows_v[:, b, :, :].reshape(512, _IN)
        ne_b = _dot(psel, er_b)
        hs.append(_leaky(_dot(ne_b, q) + qb[None, :]))

    for i in range(64):
        pltpu.make_async_copy(
            w_hbm.at[pl.ds(0, 8), pl.ds(col0, 128)], wrows_v.at[i], sem_w,
        ).wait()
    lsel = (lax.broadcasted_iota(jnp.int32, (128, 8), 0) == lane
            ).astype(jnp.float32)
    wl = _dot(wrows_v[:].reshape(512, 128), lsel)
    w64 = _dot(psel, wl[:, :1])
    denom = jnp.sum(w64) + 1e-6

    pltpu.make_async_copy(
        emb_hbm.at[:, pl.ds(0, 8), :], crow_v, sem_c,
    ).wait()
    csel = (lax.broadcasted_iota(jnp.int32, (_B, 8, 128), 1) == (nid - nrow0))
    ce = jnp.sum(jnp.where(csel, crow_v[:], 0.0), axis=1)

    ws_rows = [jnp.sum(h * w64, axis=0, keepdims=True) / denom
               for h in hs]
    wsm = jnp.concatenate(ws_rows, axis=0)
    cc = jnp.concatenate([ce, wsm], axis=1)
    cp_wk.wait()
    cp_wb.wait()
    h2 = _leaky(_dot(cc, wk_v[:]) + wb_v[:][None, :])
    nrm = jnp.sqrt(jnp.sum(h2 * h2, axis=1, keepdims=True)) + 1e-6
    o_ref[:] = h2 / nrm


def kernel(embeddings, weights, Q_kernel, Q_bias, W_kernel, W_bias,
           neighbor_set, node_id):
    ns = neighbor_set.astype(jnp.int32)
    nid1 = jnp.asarray(node_id, jnp.int32).reshape(1)
    vmem = pl.BlockSpec(memory_space=pltpu.MemorySpace.VMEM)
    hbm = pl.BlockSpec(memory_space=pltpu.MemorySpace.HBM)
    smem = pl.BlockSpec(memory_space=pltpu.MemorySpace.SMEM)
    return pl.pallas_call(
        _body,
        in_specs=[smem, smem, hbm, hbm, hbm, hbm, hbm, hbm, hbm],
        out_specs=vmem,
        out_shape=jax.ShapeDtypeStruct((_B, _OUT), jnp.float32),
        scratch_shapes=[
            pltpu.VMEM((64, _B, 8, _IN), jnp.float32),
            pltpu.VMEM((_B, 8, _IN), jnp.float32),
            pltpu.VMEM((64, 8, 128), jnp.float32),
            pltpu.VMEM((64,), jnp.int32),
            pltpu.VMEM((_IN, _HID), jnp.float32),
            pltpu.VMEM((_HID,), jnp.float32),
            pltpu.VMEM((_IN + _HID, _OUT), jnp.float32),
            pltpu.VMEM((_OUT,), jnp.float32),
            pltpu.SemaphoreType.DMA,
            pltpu.SemaphoreType.DMA,
            pltpu.SemaphoreType.DMA,
            pltpu.SemaphoreType.DMA,
            pltpu.SemaphoreType.DMA,
            pltpu.SemaphoreType.DMA,
            pltpu.SemaphoreType.DMA,
            pltpu.SemaphoreType.DMA,
        ],
    )(ns, nid1, embeddings, weights, ns, Q_kernel, Q_bias, W_kernel, W_bias)


# speedup vs baseline: 5.2890x; 1.0529x over previous
import jax
import jax.numpy as jnp
from jax import lax
from jax.experimental import pallas as pl
from jax.experimental.pallas import tpu as pltpu


_B, _N, _IN, _HID, _OUT = 4, 10000, 128, 256, 128


def _leaky(x):
    return jnp.where(x >= 0, x, 0.3 * x)


def _dot(a, b):
    return jnp.dot(a, b, preferred_element_type=jnp.float32)


def _body(ns_ref, nid_ref, emb_hbm, w_hbm, ns_hbm, q_hbm, qb_hbm,
          wk_hbm, wb_hbm, o_ref, erows_v, crow_v, wrows_v,
          nsv_v, q_v, qb_v, wk_v, wb_v,
          sem_e, sem_c, sem_w, sem_n, sem_q, sem_qb, sem_wk, sem_wb):
    nid = nid_ref[0]
    col0 = pl.multiple_of(nid & -128, 128)
    lane = nid - col0
    nrow0 = pl.multiple_of(nid & -8, 8)

    cp_ns = pltpu.make_async_copy(ns_hbm, nsv_v, sem_n)
    cp_ns.start()
    for i in range(64):
        row0 = pl.multiple_of(ns_ref[i] & -8, 8)
        pltpu.make_async_copy(
            emb_hbm.at[:, pl.ds(row0, 8), :], erows_v.at[i], sem_e,
        ).start()
        pltpu.make_async_copy(
            w_hbm.at[pl.ds(row0, 8), pl.ds(col0, 128)], wrows_v.at[i], sem_w,
        ).start()
    pltpu.make_async_copy(
        emb_hbm.at[:, pl.ds(nrow0, 8), :], crow_v, sem_c,
    ).start()
    cp_q = pltpu.make_async_copy(q_hbm, q_v, sem_q)
    cp_qb = pltpu.make_async_copy(qb_hbm, qb_v, sem_qb)
    cp_wk = pltpu.make_async_copy(wk_hbm, wk_v, sem_wk)
    cp_wb = pltpu.make_async_copy(wb_hbm, wb_v, sem_wb)
    cp_q.start(); cp_qb.start(); cp_wk.start(); cp_wb.start()

    cp_ns.wait()
    subl = nsv_v[:] % 8
    colv = lax.broadcasted_iota(jnp.int32, (64, 512), 1)
    rowv = lax.broadcasted_iota(jnp.int32, (64, 512), 0)
    psel = ((colv // 8 == rowv) & (colv % 8 == subl[:, None])
            ).astype(jnp.float32)

    for i in range(64):
        pltpu.make_async_copy(
            emb_hbm.at[:, pl.ds(0, 8), :], erows_v.at[i], sem_e,
        ).wait()
    cp_q.wait()
    cp_qb.wait()
    q = q_v[:]
    qb = qb_v[:]

    ne_rows = jnp.concatenate(
        [_dot(psel, erows_v[:, b, :, :].reshape(512, _IN)) for b in range(_B)],
        axis=0)
    h_all = _leaky(_dot(ne_rows, q) + qb[None, :])

    for i in range(64):
        pltpu.make_async_copy(
            w_hbm.at[pl.ds(0, 8), pl.ds(col0, 128)], wrows_v.at[i], sem_w,
        ).wait()
    lsel = (lax.broadcasted_iota(jnp.int32, (128, 8), 0) == lane
            ).astype(jnp.float32)
    wl = _dot(wrows_v[:].reshape(512, 128), lsel)
    w64 = _dot(psel, wl[:, :1])
    denom = jnp.sum(w64) + 1e-6

    pltpu.make_async_copy(
        emb_hbm.at[:, pl.ds(0, 8), :], crow_v, sem_c,
    ).wait()
    csel = (lax.broadcasted_iota(jnp.int32, (_B, 8, 128), 1) == (nid - nrow0))
    ce = jnp.sum(jnp.where(csel, crow_v[:], 0.0), axis=1)

    wt = jnp.concatenate([w64.reshape(1, 64)] * _B, axis=1)
    bsel = (lax.broadcasted_iota(jnp.int32, (_B, _B * 64), 1) // 64
            == lax.broadcasted_iota(jnp.int32, (_B, _B * 64), 0))
    w3 = jnp.where(bsel, wt, 0.0)
    wsm = _dot(w3, h_all) / denom
    cc = jnp.concatenate([ce, wsm], axis=1)
    cp_wk.wait()
    cp_wb.wait()
    h2 = _leaky(_dot(cc, wk_v[:]) + wb_v[:][None, :])
    nrm = jnp.sqrt(jnp.sum(h2 * h2, axis=1, keepdims=True)) + 1e-6
    o_ref[:] = h2 / nrm


def kernel(embeddings, weights, Q_kernel, Q_bias, W_kernel, W_bias,
           neighbor_set, node_id):
    ns = neighbor_set.astype(jnp.int32)
    nid1 = jnp.asarray(node_id, jnp.int32).reshape(1)
    vmem = pl.BlockSpec(memory_space=pltpu.MemorySpace.VMEM)
    hbm = pl.BlockSpec(memory_space=pltpu.MemorySpace.HBM)
    smem = pl.BlockSpec(memory_space=pltpu.MemorySpace.SMEM)
    return pl.pallas_call(
        _body,
        in_specs=[smem, smem, hbm, hbm, hbm, hbm, hbm, hbm, hbm],
        out_specs=vmem,
        out_shape=jax.ShapeDtypeStruct((_B, _OUT), jnp.float32),
        scratch_shapes=[
            pltpu.VMEM((64, _B, 8, _IN), jnp.float32),
            pltpu.VMEM((_B, 8, _IN), jnp.float32),
            pltpu.VMEM((64, 8, 128), jnp.float32),
            pltpu.VMEM((64,), jnp.int32),
            pltpu.VMEM((_IN, _HID), jnp.float32),
            pltpu.VMEM((_HID,), jnp.float32),
            pltpu.VMEM((_IN + _HID, _OUT), jnp.float32),
            pltpu.VMEM((_OUT,), jnp.float32),
            pltpu.SemaphoreType.DMA,
            pltpu.SemaphoreType.DMA,
            pltpu.SemaphoreType.DMA,
            pltpu.SemaphoreType.DMA,
            pltpu.SemaphoreType.DMA,
            pltpu.SemaphoreType.DMA,
            pltpu.SemaphoreType.DMA,
            pltpu.SemaphoreType.DMA,
        ],
    )(ns, nid1, embeddings, weights, ns, Q_kernel, Q_bias, W_kernel, W_bias)


# speedup vs baseline: 5.8806x; 1.1118x over previous
import jax
import jax.numpy as jnp
from jax import lax
from jax.experimental import pallas as pl
from jax.experimental.pallas import tpu as pltpu


_B, _N, _IN, _HID, _OUT = 4, 10000, 128, 256, 128


def _leaky(x):
    return jnp.where(x >= 0, x, 0.3 * x)


def _dot(a, b):
    return jnp.dot(a, b, preferred_element_type=jnp.float32)


def _body(ns_ref, nid_ref, emb_hbm, w_hbm, q_hbm, qb_hbm,
          wk_hbm, wb_hbm, o_ref, erows_v, crow_v, wrows_v,
          q_v, qb_v, wk_v, wb_v,
          sem_e, sem_c, sem_w, sem_q, sem_qb, sem_wk, sem_wb):
    nid = nid_ref[0]
    col0 = pl.multiple_of(nid & -128, 128)
    lane = nid - col0

    for i in range(64):
        pltpu.make_async_copy(
            emb_hbm.at[:, pl.ds(ns_ref[i], 1), :], erows_v.at[:, pl.ds(i, 1)],
            sem_e,
        ).start()
    cp_q = pltpu.make_async_copy(q_hbm, q_v, sem_q)
    cp_qb = pltpu.make_async_copy(qb_hbm, qb_v, sem_qb)
    cp_q.start(); cp_qb.start()
    for i in range(64):
        pltpu.make_async_copy(
            w_hbm.at[pl.ds(ns_ref[i], 1), pl.ds(col0, 128)],
            wrows_v.at[pl.ds(i, 1)], sem_w,
        ).start()
    pltpu.make_async_copy(
        emb_hbm.at[:, pl.ds(nid, 1), :], crow_v, sem_c,
    ).start()
    cp_wk = pltpu.make_async_copy(wk_hbm, wk_v, sem_wk)
    cp_wb = pltpu.make_async_copy(wb_hbm, wb_v, sem_wb)
    cp_wk.start(); cp_wb.start()

    for i in range(64):
        pltpu.make_async_copy(
            emb_hbm.at[:, pl.ds(0, 1), :], erows_v.at[:, pl.ds(i, 1)], sem_e,
        ).wait()
    cp_q.wait()
    cp_qb.wait()
    q = q_v[:]
    qb = qb_v[:]

    ne_rows = erows_v[:].reshape(_B * 64, _IN)
    h_all = _leaky(_dot(ne_rows, q) + qb[None, :])

    for i in range(64):
        pltpu.make_async_copy(
            w_hbm.at[pl.ds(0, 1), pl.ds(col0, 128)],
            wrows_v.at[pl.ds(i, 1)], sem_w,
        ).wait()
    lsel = (lax.broadcasted_iota(jnp.int32, (128, 1), 0) == lane
            ).astype(jnp.float32)
    w64 = _dot(wrows_v[:], lsel)
    denom = jnp.sum(w64) + 1e-6

    pltpu.make_async_copy(
        emb_hbm.at[:, pl.ds(0, 1), :], crow_v, sem_c,
    ).wait()
    ce = crow_v[:].reshape(_B, _IN)

    wt = jnp.concatenate([w64.reshape(1, 64)] * _B, axis=1)
    bsel = (lax.broadcasted_iota(jnp.int32, (_B, _B * 64), 1) // 64
            == lax.broadcasted_iota(jnp.int32, (_B, _B * 64), 0))
    w3 = jnp.where(bsel, wt, 0.0)
    wsm = _dot(w3, h_all) / denom
    cc = jnp.concatenate([ce, wsm], axis=1)
    cp_wk.wait()
    cp_wb.wait()
    h2 = _leaky(_dot(cc, wk_v[:]) + wb_v[:][None, :])
    nrm = jnp.sqrt(jnp.sum(h2 * h2, axis=1, keepdims=True)) + 1e-6
    o_ref[:] = h2 / nrm


def kernel(embeddings, weights, Q_kernel, Q_bias, W_kernel, W_bias,
           neighbor_set, node_id):
    ns = neighbor_set.astype(jnp.int32)
    nid1 = jnp.asarray(node_id, jnp.int32).reshape(1)
    vmem = pl.BlockSpec(memory_space=pltpu.MemorySpace.VMEM)
    hbm = pl.BlockSpec(memory_space=pltpu.MemorySpace.HBM)
    smem = pl.BlockSpec(memory_space=pltpu.MemorySpace.SMEM)
    return pl.pallas_call(
        _body,
        in_specs=[smem, smem, hbm, hbm, hbm, hbm, hbm, hbm],
        out_specs=vmem,
        out_shape=jax.ShapeDtypeStruct((_B, _OUT), jnp.float32),
        scratch_shapes=[
            pltpu.VMEM((_B, 64, _IN), jnp.float32),
            pltpu.VMEM((_B, 1, _IN), jnp.float32),
            pltpu.VMEM((64, 128), jnp.float32),
            pltpu.VMEM((_IN, _HID), jnp.float32),
            pltpu.VMEM((_HID,), jnp.float32),
            pltpu.VMEM((_IN + _HID, _OUT), jnp.float32),
            pltpu.VMEM((_OUT,), jnp.float32),
            pltpu.SemaphoreType.DMA,
            pltpu.SemaphoreType.DMA,
            pltpu.SemaphoreType.DMA,
            pltpu.SemaphoreType.DMA,
            pltpu.SemaphoreType.DMA,
            pltpu.SemaphoreType.DMA,
            pltpu.SemaphoreType.DMA,
        ],
    )(ns, nid1, embeddings, weights, Q_kernel, Q_bias, W_kernel, W_bias)
